# baseline (speedup 1.0000x reference)
import math

import jax
import jax.numpy as jnp
from jax import lax
from jax.experimental import pallas as pl
from jax.experimental.pallas import tpu as pltpu

N = 1024
IN_DIM = 128
HID = 128
BC = 128
NB = N // BC


def _attend(k, q_blk, nm_blk, v, s_blk):
    logits = lax.dot_general(k, q_blk, (((1,), (1,)), ((), ())),
                             preferred_element_type=jnp.float32)
    masked = logits + nm_blk
    amax = jnp.max(masked, axis=0)
    amax = jnp.where(jnp.isfinite(amax), amax, 0.0)
    ex = jnp.exp(masked - amax[None, :])
    denom = jnp.sum(ex, axis=0)
    agg = lax.dot_general(ex, v, (((0,), (0,)), ((), ())),
                          preferred_element_type=jnp.float32)
    return agg * (1.0 / (denom[:, None] + 1e-16)) + s_blk


def _block_kernel(x_ref, adj_ref,
                  wq1, bq1, wk1, bk1, wv1, bv1, ws1, bs1,
                  wq2, bq2, wk2, bk2, wv2, bv2, ws2, bs2,
                  out_ref,
                  q1_s, k1_s, v1_s, s1_s, h1_s, nm_s):
    j = pl.program_id(0)
    scale = 1.0 / math.sqrt(float(HID))

    @pl.when(j == 0)
    def _projections():
        x = x_ref[:]
        q1_s[:] = jnp.dot(x, wq1[:] * scale,
                          preferred_element_type=jnp.float32) + bq1[:] * scale
        k1_s[:] = jnp.dot(x, wk1[:], preferred_element_type=jnp.float32) + bk1[:]
        v1_s[:] = jnp.dot(x, wv1[:], preferred_element_type=jnp.float32) + bv1[:]
        s1_s[:] = jnp.dot(x, ws1[:], preferred_element_type=jnp.float32) + bs1[:]

    @pl.when(j < NB)
    def _layer1_block():
        c0 = j * BC
        nm = jnp.where(adj_ref[:] != 0.0, 0.0, -jnp.inf)
        nm_s[:, pl.ds(c0, BC)] = nm
        h = _attend(k1_s[:], q1_s[pl.ds(c0, BC), :], nm,
                    v1_s[:], s1_s[pl.ds(c0, BC), :])
        h1_s[pl.ds(c0, BC), :] = jnp.where(
            h > 0.0, h, jnp.exp(jnp.minimum(h, 0.0)) - 1.0)

    @pl.when(j == NB)
    def _layer2():
        h1 = h1_s[:]
        sc2 = 1.0 / math.sqrt(float(IN_DIM))
        q2 = jnp.dot(h1, wq2[:] * sc2,
                     preferred_element_type=jnp.float32) + bq2[:] * sc2
        k2 = jnp.dot(h1, wk2[:], preferred_element_type=jnp.float32) + bk2[:]
        v2 = jnp.dot(h1, wv2[:], preferred_element_type=jnp.float32) + bv2[:]
        s2 = jnp.dot(h1, ws2[:], preferred_element_type=jnp.float32) + bs2[:]
        out_ref[:] = _attend(k2, q2, nm_s[:], v2, s2)


@jax.jit
def kernel(x, XY_Adj, Wq1, bq1, Wk1, bk1, Wv1, bv1, Ws1, bs1,
           Wq2, bq2, Wk2, bk2, Wv2, bv2, Ws2, bs2):
    biases = [b.reshape(1, -1) for b in (bq1, bk1, bv1, bs1, bq2, bk2, bv2, bs2)]
    bq1, bk1, bv1, bs1, bq2, bk2, bv2, bs2 = biases
    full = lambda shape: pl.BlockSpec(shape, lambda j: (0, 0))
    f32 = jnp.float32
    return pl.pallas_call(
        _block_kernel,
        grid=(NB + 1,),
        in_specs=[
            full((N, IN_DIM)),
            pl.BlockSpec((N, BC), lambda j: (0, jnp.minimum(j, NB - 1))),
            full((IN_DIM, HID)), full((1, HID)),
            full((IN_DIM, HID)), full((1, HID)),
            full((IN_DIM, HID)), full((1, HID)),
            full((IN_DIM, HID)), full((1, HID)),
            full((HID, IN_DIM)), full((1, IN_DIM)),
            full((HID, IN_DIM)), full((1, IN_DIM)),
            full((HID, IN_DIM)), full((1, IN_DIM)),
            full((HID, IN_DIM)), full((1, IN_DIM)),
        ],
        out_specs=full((N, IN_DIM)),
        out_shape=jax.ShapeDtypeStruct((N, IN_DIM), f32),
        scratch_shapes=[
            pltpu.VMEM((N, HID), f32),
            pltpu.VMEM((N, HID), f32),
            pltpu.VMEM((N, HID), f32),
            pltpu.VMEM((N, HID), f32),
            pltpu.VMEM((N, HID), f32),
            pltpu.VMEM((N, N), f32),
        ],
    )(x, XY_Adj,
      Wq1, bq1, Wk1, bk1, Wv1, bv1, Ws1, bs1,
      Wq2, bq2, Wk2, bk2, Wv2, bv2, Ws2, bs2)


# speedup vs baseline: 1.3131x; 1.3131x over previous
import math

import jax
import jax.numpy as jnp
from jax import lax
from jax.experimental import pallas as pl
from jax.experimental.pallas import tpu as pltpu

N = 1024
IN_DIM = 128
HID = 128
BC = 512
NB = N // BC


def _attend(k, q_blk, nm_blk, v, s_blk):
    logits = lax.dot_general(k, q_blk, (((1,), (1,)), ((), ())),
                             preferred_element_type=jnp.float32)
    masked = logits + nm_blk
    amax = jnp.max(masked, axis=0)
    amax = jnp.where(jnp.isfinite(amax), amax, 0.0)
    ex = jnp.exp(masked - amax[None, :])
    denom = jnp.sum(ex, axis=0)
    agg = lax.dot_general(ex, v, (((0,), (0,)), ((), ())),
                          preferred_element_type=jnp.float32)
    return agg * (1.0 / (denom[:, None] + 1e-16)) + s_blk


def _block_kernel(x_ref, adj_ref,
                  wq1, bq1, wk1, bk1, wv1, bv1, ws1, bs1,
                  wq2, bq2, wk2, bk2, wv2, bv2, ws2, bs2,
                  out_ref,
                  q1_s, k1_s, v1_s, s1_s, h1_s, nm_s):
    j = pl.program_id(0)
    scale = 1.0 / math.sqrt(float(HID))

    @pl.when(j == 0)
    def _projections():
        x = x_ref[:]
        q1_s[:] = jnp.dot(x, wq1[:] * scale,
                          preferred_element_type=jnp.float32) + bq1[:] * scale
        k1_s[:] = jnp.dot(x, wk1[:], preferred_element_type=jnp.float32) + bk1[:]
        v1_s[:] = jnp.dot(x, wv1[:], preferred_element_type=jnp.float32) + bv1[:]
        s1_s[:] = jnp.dot(x, ws1[:], preferred_element_type=jnp.float32) + bs1[:]

    @pl.when(j < NB)
    def _layer1_block():
        c0 = j * BC
        nm = jnp.where(adj_ref[:] != 0.0, 0.0, -jnp.inf)
        nm_s[:, pl.ds(c0, BC)] = nm
        h = _attend(k1_s[:], q1_s[pl.ds(c0, BC), :], nm,
                    v1_s[:], s1_s[pl.ds(c0, BC), :])
        h1_s[pl.ds(c0, BC), :] = jnp.where(
            h > 0.0, h, jnp.exp(jnp.minimum(h, 0.0)) - 1.0)

    @pl.when(j == NB)
    def _layer2():
        h1 = h1_s[:]
        sc2 = 1.0 / math.sqrt(float(IN_DIM))
        q2 = jnp.dot(h1, wq2[:] * sc2,
                     preferred_element_type=jnp.float32) + bq2[:] * sc2
        k2 = jnp.dot(h1, wk2[:], preferred_element_type=jnp.float32) + bk2[:]
        v2 = jnp.dot(h1, wv2[:], preferred_element_type=jnp.float32) + bv2[:]
        s2 = jnp.dot(h1, ws2[:], preferred_element_type=jnp.float32) + bs2[:]
        out_ref[:] = _attend(k2, q2, nm_s[:], v2, s2)


@jax.jit
def kernel(x, XY_Adj, Wq1, bq1, Wk1, bk1, Wv1, bv1, Ws1, bs1,
           Wq2, bq2, Wk2, bk2, Wv2, bv2, Ws2, bs2):
    biases = [b.reshape(1, -1) for b in (bq1, bk1, bv1, bs1, bq2, bk2, bv2, bs2)]
    bq1, bk1, bv1, bs1, bq2, bk2, bv2, bs2 = biases
    full = lambda shape: pl.BlockSpec(shape, lambda j: (0, 0))
    f32 = jnp.float32
    return pl.pallas_call(
        _block_kernel,
        grid=(NB + 1,),
        in_specs=[
            full((N, IN_DIM)),
            pl.BlockSpec((N, BC), lambda j: (0, jnp.minimum(j, NB - 1))),
            full((IN_DIM, HID)), full((1, HID)),
            full((IN_DIM, HID)), full((1, HID)),
            full((IN_DIM, HID)), full((1, HID)),
            full((IN_DIM, HID)), full((1, HID)),
            full((HID, IN_DIM)), full((1, IN_DIM)),
            full((HID, IN_DIM)), full((1, IN_DIM)),
            full((HID, IN_DIM)), full((1, IN_DIM)),
            full((HID, IN_DIM)), full((1, IN_DIM)),
        ],
        out_specs=full((N, IN_DIM)),
        out_shape=jax.ShapeDtypeStruct((N, IN_DIM), f32),
        scratch_shapes=[
            pltpu.VMEM((N, HID), f32),
            pltpu.VMEM((N, HID), f32),
            pltpu.VMEM((N, HID), f32),
            pltpu.VMEM((N, HID), f32),
            pltpu.VMEM((N, HID), f32),
            pltpu.VMEM((N, N), f32),
        ],
    )(x, XY_Adj,
      Wq1, bq1, Wk1, bk1, Wv1, bv1, Ws1, bs1,
      Wq2, bq2, Wk2, bk2, Wv2, bv2, Ws2, bs2)


# speedup vs baseline: 1.3991x; 1.0655x over previous
import math

import jax
import jax.numpy as jnp
from jax import lax
from jax.experimental import pallas as pl

N = 1024
IN_DIM = 128
HID = 128


def _layer(x, neg_mask, Wq, bq, Wk, bk, Wv, bv, Ws, bs):
    scale = 1.0 / math.sqrt(float(Wq.shape[1]))
    Wq = Wq * scale
    bq = bq * scale
    q = jnp.dot(x, Wq, preferred_element_type=jnp.float32) + bq
    k = jnp.dot(x, Wk, preferred_element_type=jnp.float32) + bk
    v = jnp.dot(x, Wv, preferred_element_type=jnp.float32) + bv
    s = jnp.dot(x, Ws, preferred_element_type=jnp.float32) + bs
    logits = lax.dot_general(k, q, (((1,), (1,)), ((), ())),
                             preferred_element_type=jnp.float32)
    masked = logits + neg_mask
    amax = jnp.max(masked, axis=0)
    amax = jnp.where(jnp.isfinite(amax), amax, 0.0)
    ex = jnp.exp(masked - amax[None, :])
    denom = jnp.sum(ex, axis=0)
    agg = lax.dot_general(ex, v, (((0,), (0,)), ((), ())),
                          preferred_element_type=jnp.float32)
    out = agg * (1.0 / (denom[:, None] + 1e-16))
    return out + s


def _block_kernel(x_ref, adj_ref,
                  wq1, bq1, wk1, bk1, wv1, bv1, ws1, bs1,
                  wq2, bq2, wk2, bk2, wv2, bv2, ws2, bs2,
                  out_ref):
    x = x_ref[:]
    neg_mask = jnp.where(adj_ref[:] != 0.0, 0.0, -jnp.inf)
    h1 = _layer(x, neg_mask,
                wq1[:], bq1[:], wk1[:], bk1[:], wv1[:], bv1[:], ws1[:], bs1[:])
    h1 = jnp.where(h1 > 0.0, h1, jnp.exp(jnp.minimum(h1, 0.0)) - 1.0)
    out_ref[:] = _layer(h1, neg_mask,
                        wq2[:], bq2[:], wk2[:], bk2[:], wv2[:], bv2[:],
                        ws2[:], bs2[:])


@jax.jit
def kernel(x, XY_Adj, Wq1, bq1, Wk1, bk1, Wv1, bv1, Ws1, bs1,
           Wq2, bq2, Wk2, bk2, Wv2, bv2, Ws2, bs2):
    return pl.pallas_call(
        _block_kernel,
        out_shape=jax.ShapeDtypeStruct((N, IN_DIM), jnp.float32),
    )(x, XY_Adj,
      Wq1, bq1, Wk1, bk1, Wv1, bv1, Ws1, bs1,
      Wq2, bq2, Wk2, bk2, Wv2, bv2, Ws2, bs2)
